# baseline (speedup 1.0000x reference)
import functools

import jax
import jax.numpy as jnp
from jax import lax
from jax.experimental import pallas as pl
from jax.experimental.pallas import tpu as pltpu
from jax.experimental.pallas import tpu_sc as plsc

NC = 2
NS = 16
NW = NC * NS
LANES = 16
CHUNK = 128


def _compose_w(bases, comp, blk=400):
    B, N, D = bases.shape
    R = comp.shape[0]
    assert N % blk == 0

    def body(comp_ref, bases_ref, w_ref):
        for r in range(R):
            acc = comp_ref[r, 0] * bases_ref[0]
            for b in range(1, B):
                acc = acc + comp_ref[r, b] * bases_ref[b]
            w_ref[r] = acc

    return pl.pallas_call(
        body,
        grid=(N // blk,),
        in_specs=[
            pl.BlockSpec(memory_space=pltpu.SMEM),
            pl.BlockSpec((B, blk, D), lambda i: (0, i, 0)),
        ],
        out_specs=pl.BlockSpec((R, blk, D), lambda i: (0, i, 0)),
        out_shape=jax.ShapeDtypeStruct((R, N, D), jnp.float32),
    )(comp, bases)


def _finalize(pagg, pdeg, root_pad, bias2d, n_acc, d, blk=512):
    def body(p_ref, deg_ref, root_ref, bias_ref, out_ref):
        s = p_ref[0] + p_ref[1]
        dg = deg_ref[0] + deg_ref[1]
        out_ref[...] = s / jnp.maximum(dg, 1.0) + root_ref[...] + bias_ref[...]

    return pl.pallas_call(
        body,
        grid=(n_acc // blk,),
        in_specs=[
            pl.BlockSpec((NC, blk, d), lambda i: (0, i, 0)),
            pl.BlockSpec((NC, blk, 1), lambda i: (0, i, 0)),
            pl.BlockSpec((blk, d), lambda i: (i, 0)),
            pl.BlockSpec((1, d), lambda i: (0, 0)),
        ],
        out_specs=pl.BlockSpec((blk, d), lambda i: (i, 0)),
        out_shape=jax.ShapeDtypeStruct((n_acc, d), jnp.float32),
    )(pagg.reshape(NC, n_acc, d), pdeg.reshape(NC, n_acc, 1), root_pad, bias2d)


def _make_sc_agg(n_nodes, n_acc, d, ept, kch):
    npt = n_acc // NS
    mesh = plsc.VectorSubcoreMesh(
        core_axis_name="c", subcore_axis_name="s", num_cores=NC, num_subcores=NS)

    @functools.partial(
        pl.kernel,
        out_type=(
            jax.ShapeDtypeStruct((NC * n_acc, d), jnp.float32),
            jax.ShapeDtypeStruct((NC * n_acc,), jnp.float32),
        ),
        mesh=mesh,
        scratch_types=[
            pltpu.VMEM((ept,), jnp.int32),
            pltpu.VMEM((2, CHUNK), jnp.int32),
            pltpu.VMEM((2, CHUNK), jnp.int32),
            pltpu.VMEM((2, CHUNK, d), jnp.float32),
            pltpu.VMEM((CHUNK,), jnp.float32),
            pltpu.VMEM((npt,), jnp.float32),
            pltpu.VMEM_SHARED((n_acc, d), jnp.float32),
            pltpu.VMEM_SHARED((n_acc,), jnp.float32),
            [pltpu.SemaphoreType.DMA] * 2,
            [pltpu.SemaphoreType.DMA] * 2,
            [pltpu.SemaphoreType.DMA] * 2,
            [pltpu.SemaphoreType.DMA] * 2,
        ],
    )
    def sc_agg(w_hbm, packed_hbm, pagg_hbm, pdeg_hbm,
               packed_v, flat_v, dst_v, rows_v, ones_v, degbuf_v,
               acc_sh, deg_sh, gsem, ssem, osem, hsem):
        c = lax.axis_index("c")
        s = lax.axis_index("s")
        wid = c * NS + s
        row0 = s * npt

        def zrow(i, carry):
            for j in range(d // LANES):
                rows_v[0, i, pl.ds(j * LANES, LANES)] = jnp.zeros((LANES,), jnp.float32)
            return carry
        lax.fori_loop(0, CHUNK, zrow, 0)
        for j in range(CHUNK // LANES):
            ones_v[pl.ds(j * LANES, LANES)] = jnp.ones((LANES,), jnp.float32)

        def zdeg(i, carry):
            degbuf_v[pl.ds(i * LANES, LANES)] = jnp.zeros((LANES,), jnp.float32)
            return carry
        lax.fori_loop(0, npt // LANES, zdeg, 0)

        for t in range(npt // CHUNK):
            pltpu.sync_copy(rows_v.at[0], acc_sh.at[pl.ds(row0 + t * CHUNK, CHUNK)])
        pltpu.sync_copy(degbuf_v, deg_sh.at[pl.ds(row0, npt)])
        plsc.subcore_barrier()

        pltpu.sync_copy(packed_hbm.at[pl.ds(wid * ept, ept)], packed_v)

        def unpack(k, b):
            for j in range(CHUNK // LANES):
                p16 = packed_v[pl.ds(k * CHUNK + j * LANES, LANES)]
                flat_v[b, pl.ds(j * LANES, LANES)] = p16 & 0x1FFFF
                dst_v[b, pl.ds(j * LANES, LANES)] = p16 >> 17
            return None

        def start_gather(k, b):
            unpack(k, b)
            pltpu.async_copy(w_hbm.at[flat_v.at[b]], rows_v.at[b], gsem[b])

        def wait_gather(b):
            pltpu.make_async_copy(w_hbm.at[flat_v.at[b]], rows_v.at[b], gsem[b]).wait()

        def start_scatter(b):
            pltpu.async_copy(rows_v.at[b], acc_sh.at[dst_v.at[b]], ssem[b], add=True)
            pltpu.async_copy(ones_v, deg_sh.at[dst_v.at[b]], osem[b], add=True)

        def wait_scatter(b):
            pltpu.make_async_copy(rows_v.at[b], acc_sh.at[dst_v.at[b]], ssem[b]).wait()
            pltpu.make_async_copy(ones_v, deg_sh.at[dst_v.at[b]], osem[b]).wait()

        for b in (0, 1):
            start_gather(b, b)
            wait_gather(b)
            start_scatter(b)

        def step(i, carry):
            k0 = i * 2
            for b in (0, 1):
                wait_scatter(b)
                start_gather(k0 + b, b)
                wait_gather(b)
                start_scatter(b)
            return carry
        lax.fori_loop(1, kch // 2, step, 0)

        wait_scatter(0)
        wait_scatter(1)
        plsc.subcore_barrier()

        out0 = c * n_acc + row0
        nout = npt // CHUNK
        for t in range(nout):
            b = t % 2
            if t >= 2:
                prev = out0 + (t - 2) * CHUNK
                pltpu.make_async_copy(
                    rows_v.at[b], pagg_hbm.at[pl.ds(prev, CHUNK)], hsem[b]).wait()
            pltpu.sync_copy(acc_sh.at[pl.ds(row0 + t * CHUNK, CHUNK)], rows_v.at[b])
            pltpu.async_copy(
                rows_v.at[b], pagg_hbm.at[pl.ds(out0 + t * CHUNK, CHUNK)], hsem[b])
        pltpu.sync_copy(deg_sh.at[pl.ds(row0, npt)], degbuf_v)
        pltpu.sync_copy(degbuf_v, pdeg_hbm.at[pl.ds(out0, npt)])
        for t in (nout - 2, nout - 1):
            b = t % 2
            pltpu.make_async_copy(
                rows_v.at[b], pagg_hbm.at[pl.ds(out0 + t * CHUNK, CHUNK)], hsem[b]).wait()

    return sc_agg


def kernel(edge_index, edge_type, bases, comp, root, bias):
    B, N, D = bases.shape
    R = comp.shape[0]
    E = edge_type.shape[0]

    kch = -(-E // (NW * CHUNK))
    kch = kch + (kch % 2)
    ept = kch * CHUNK
    e_pad = NW * ept
    n_acc = ((N + NS * CHUNK - 1) // (NS * CHUNK)) * (NS * CHUNK)

    src = edge_index[0]
    dst = edge_index[1]
    pad = e_pad - E
    flat = edge_type * N + src
    packed = (dst << 17) | flat
    packed = jnp.concatenate(
        [packed, jnp.full((pad,), N << 17, packed.dtype)])

    w = _compose_w(bases, comp)
    w2d = w.reshape(R * N, D)

    sc_agg = _make_sc_agg(N, n_acc, D, ept, kch)
    pagg, pdeg = sc_agg(w2d, packed)

    root_pad = jnp.pad(root, ((0, n_acc - N), (0, 0)))
    out = _finalize(pagg, pdeg, root_pad, bias.reshape(1, D), n_acc, D)
    return out[:N]


# speedup vs baseline: 1.3504x; 1.3504x over previous
import functools

import jax
import jax.numpy as jnp
from jax import lax
from jax.experimental import pallas as pl
from jax.experimental.pallas import tpu as pltpu
from jax.experimental.pallas import tpu_sc as plsc

NC = 2
NS = 16
NW = NC * NS
LANES = 16
CHUNK = 128


def _compose_w(bases, comp, blk=400):
    B, N, D = bases.shape
    R = comp.shape[0]
    assert N % blk == 0

    def body(comp_ref, bases_ref, w_ref):
        for r in range(R):
            acc = comp_ref[r, 0] * bases_ref[0]
            for b in range(1, B):
                acc = acc + comp_ref[r, b] * bases_ref[b]
            w_ref[r] = acc

    return pl.pallas_call(
        body,
        grid=(N // blk,),
        in_specs=[
            pl.BlockSpec(memory_space=pltpu.SMEM),
            pl.BlockSpec((B, blk, D), lambda i: (0, i, 0)),
        ],
        out_specs=pl.BlockSpec((R, blk, D), lambda i: (0, i, 0)),
        out_shape=jax.ShapeDtypeStruct((R, N, D), jnp.float32),
    )(comp, bases)


def _finalize(pagg, pdeg, root_pad, bias2d, n_acc, d, blk=512):
    def body(p_ref, deg_ref, root_ref, bias_ref, out_ref):
        s = p_ref[0] + p_ref[1]
        dg = deg_ref[0] + deg_ref[1]
        out_ref[...] = s / jnp.maximum(dg, 1.0) + root_ref[...] + bias_ref[...]

    return pl.pallas_call(
        body,
        grid=(n_acc // blk,),
        in_specs=[
            pl.BlockSpec((NC, blk, d), lambda i: (0, i, 0)),
            pl.BlockSpec((NC, blk, 1), lambda i: (0, i, 0)),
            pl.BlockSpec((blk, d), lambda i: (i, 0)),
            pl.BlockSpec((1, d), lambda i: (0, 0)),
        ],
        out_specs=pl.BlockSpec((blk, d), lambda i: (i, 0)),
        out_shape=jax.ShapeDtypeStruct((n_acc, d), jnp.float32),
    )(pagg.reshape(NC, n_acc, d), pdeg.reshape(NC, n_acc, 1), root_pad, bias2d)


def _make_sc_agg(n_nodes, n_acc, d, ept, kch):
    npt = n_acc // NS
    mesh = plsc.VectorSubcoreMesh(
        core_axis_name="c", subcore_axis_name="s", num_cores=NC, num_subcores=NS)

    @functools.partial(
        pl.kernel,
        out_type=(
            jax.ShapeDtypeStruct((NC * n_acc, d), jnp.float32),
            jax.ShapeDtypeStruct((NC * n_acc,), jnp.float32),
        ),
        mesh=mesh,
        scratch_types=[
            pltpu.VMEM((ept,), jnp.int32),
            pltpu.VMEM((ept,), jnp.int32),
            pltpu.VMEM((kch, CHUNK), jnp.int32),
            pltpu.VMEM((CHUNK, d), jnp.float32),
            pltpu.VMEM((CHUNK,), jnp.float32),
            pltpu.VMEM((npt,), jnp.float32),
            pltpu.VMEM_SHARED((n_acc, d), jnp.float32),
            pltpu.VMEM_SHARED((n_acc,), jnp.float32),
            pltpu.SemaphoreType.DMA,
            pltpu.SemaphoreType.DMA,
        ],
    )
    def sc_agg(w_hbm, src_hbm, etype_hbm, dst_hbm, pagg_hbm, pdeg_hbm,
               flat_v, etype_v, dst_v, rows_v, ones_v, degbuf_v,
               acc_sh, deg_sh, gsem, osem):
        c = lax.axis_index("c")
        s = lax.axis_index("s")
        wid = c * NS + s
        row0 = s * npt

        def zrow(i, carry):
            for j in range(d // LANES):
                rows_v[i, pl.ds(j * LANES, LANES)] = jnp.zeros((LANES,), jnp.float32)
            return carry
        lax.fori_loop(0, CHUNK, zrow, 0)
        for j in range(CHUNK // LANES):
            ones_v[pl.ds(j * LANES, LANES)] = jnp.ones((LANES,), jnp.float32)

        def zdeg(i, carry):
            degbuf_v[pl.ds(i * LANES, LANES)] = jnp.zeros((LANES,), jnp.float32)
            return carry
        lax.fori_loop(0, npt // LANES, zdeg, 0)

        for t in range(npt // CHUNK):
            pltpu.sync_copy(rows_v, acc_sh.at[pl.ds(row0 + t * CHUNK, CHUNK)])
        pltpu.sync_copy(degbuf_v, deg_sh.at[pl.ds(row0, npt)])
        plsc.subcore_barrier()

        base = wid * ept
        pltpu.sync_copy(src_hbm.at[pl.ds(base, ept)], flat_v)
        pltpu.sync_copy(etype_hbm.at[pl.ds(base, ept)], etype_v)
        pltpu.sync_copy(dst_hbm.at[wid], dst_v)

        def fl(i, carry):
            sl = pl.ds(i * LANES, LANES)
            flat_v[sl] = etype_v[sl] * n_nodes + flat_v[sl]
            return carry
        lax.fori_loop(0, ept // LANES, fl, 0)

        def step(k, carry):
            idx = flat_v.at[pl.ds(k * CHUNK, CHUNK)]
            pltpu.async_copy(w_hbm.at[idx], rows_v, gsem).wait()
            pltpu.sync_copy(rows_v, acc_sh.at[dst_v.at[k]], add=True)
            pltpu.async_copy(ones_v, deg_sh.at[dst_v.at[k]], osem, add=True)
            return carry
        lax.fori_loop(0, kch, step, 0)

        def ddrain(k, carry):
            pltpu.make_async_copy(ones_v, deg_sh.at[dst_v.at[k]], osem).wait()
            return carry
        lax.fori_loop(0, kch, ddrain, 0)
        plsc.subcore_barrier()

        out0 = c * n_acc + row0
        for t in range(npt // CHUNK):
            pltpu.sync_copy(acc_sh.at[pl.ds(row0 + t * CHUNK, CHUNK)], rows_v)
            pltpu.sync_copy(rows_v, pagg_hbm.at[pl.ds(out0 + t * CHUNK, CHUNK)])
        pltpu.sync_copy(deg_sh.at[pl.ds(row0, npt)], degbuf_v)
        pltpu.sync_copy(degbuf_v, pdeg_hbm.at[pl.ds(out0, npt)])

    return sc_agg


def kernel(edge_index, edge_type, bases, comp, root, bias):
    B, N, D = bases.shape
    R = comp.shape[0]
    E = edge_type.shape[0]

    kch = -(-E // (NW * CHUNK))
    ept = kch * CHUNK
    e_pad = NW * ept
    n_acc = ((N + NS * CHUNK - 1) // (NS * CHUNK)) * (NS * CHUNK)

    src = edge_index[0]
    dst = edge_index[1]
    pad = e_pad - E
    src_p = jnp.concatenate([src, jnp.zeros((pad,), src.dtype)])
    et_p = jnp.concatenate([edge_type, jnp.zeros((pad,), edge_type.dtype)])
    dst_p = jnp.concatenate([dst, jnp.full((pad,), N, dst.dtype)])
    dst3d = dst_p.reshape(NW, kch, CHUNK)

    w = _compose_w(bases, comp)
    w2d = w.reshape(R * N, D)

    sc_agg = _make_sc_agg(N, n_acc, D, ept, kch)
    pagg, pdeg = sc_agg(w2d, src_p, et_p, dst3d)

    root_pad = jnp.pad(root, ((0, n_acc - N), (0, 0)))
    out = _finalize(pagg, pdeg, root_pad, bias.reshape(1, D), n_acc, D)
    return out[:N]
